# baseline (speedup 1.0000x reference)
import jax
import jax.numpy as jnp
from jax.experimental import pallas as pl
from jax.experimental.pallas import tpu as pltpu

_LANES = 128
_SUBLANES = 8


def _colwise_kernel(scal_ref, xt_ref, w_ref, u_ref, out_ref, ld_ref):
    b = scal_ref[0]
    wtu = scal_ref[1]

    xf = xt_ref[...].astype(jnp.float32)
    lin = jnp.sum(xf * w_ref[...], axis=0, keepdims=True) + b
    h = jnp.tanh(lin)
    out_ref[...] = (xf + u_ref[...] * h).astype(out_ref.dtype)
    ld_ref[...] = jnp.log(jnp.abs(1.0 + (1.0 - h * h) * wtu))


def _rowwise_kernel(scal_ref, x_ref, w_ref, u_ref, out_ref, ld_ref):
    b = scal_ref[0]
    wtu = scal_ref[1]

    xf = x_ref[...].astype(jnp.float32)
    lin = jnp.sum(xf * w_ref[...], axis=-1, keepdims=True) + b
    h = jnp.tanh(lin)
    out_ref[...] = (xf + h * u_ref[...]).astype(out_ref.dtype)
    ld_ref[...] = jnp.log(jnp.abs(1.0 + (1.0 - h * h) * wtu))


def _u_hat_scalars(u, w, b, d):
    wf = w.astype(jnp.float32).reshape(1, d)
    uf = u.astype(jnp.float32).reshape(1, d)
    wtu = jnp.sum(wf * uf)
    m_wtu = -1.0 + jnp.log1p(jnp.exp(wtu))
    u_hat = jnp.where(wtu < 1.0, uf + (m_wtu - wtu) * wf / jnp.sum(wf * wf), uf)
    wtu_hat = jnp.sum(wf * u_hat)
    scalars = jnp.stack([jnp.reshape(b, ()).astype(jnp.float32), wtu_hat])
    return wf, u_hat, scalars


def kernel(x, u, w, b):
    N, d = x.shape
    wf, u_hat, scalars = _u_hat_scalars(u, w, b, d)

    if d % _SUBLANES == 0 and N % _LANES == 0:
        xt = x.T
        w_col = wf.reshape(d, 1)
        u_col = u_hat.reshape(d, 1)

        tl = 32768
        while tl > _LANES and N % tl != 0:
            tl //= 2
        grid = (N // tl,)
        block_bytes = d * tl * 8 + tl * 4
        vmem_limit = int(min(60000 * 1024, 2 * block_bytes + (16 << 20)))

        out_t, ld_row = pl.pallas_call(
            _colwise_kernel,
            out_shape=(
                jax.ShapeDtypeStruct((d, N), x.dtype),
                jax.ShapeDtypeStruct((1, N), jnp.float32),
            ),
            grid_spec=pltpu.PrefetchScalarGridSpec(
                num_scalar_prefetch=0,
                grid=grid,
                in_specs=[
                    pl.BlockSpec(memory_space=pltpu.MemorySpace.SMEM),
                    pl.BlockSpec((d, tl), lambda i: (0, i)),
                    pl.BlockSpec((d, 1), lambda i: (0, 0)),
                    pl.BlockSpec((d, 1), lambda i: (0, 0)),
                ],
                out_specs=[
                    pl.BlockSpec((d, tl), lambda i: (0, i)),
                    pl.BlockSpec((1, tl), lambda i: (0, i)),
                ],
            ),
            compiler_params=pltpu.CompilerParams(
                dimension_semantics=("parallel",),
                vmem_limit_bytes=vmem_limit),
        )(scalars, xt, w_col, u_col)

        return out_t.T, ld_row.reshape(N, 1)

    tile = min(N, 4096)
    tile = max(_SUBLANES, (tile // _SUBLANES) * _SUBLANES)
    grid = (pl.cdiv(N, tile),)
    out, ld = pl.pallas_call(
        _rowwise_kernel,
        out_shape=(
            jax.ShapeDtypeStruct((N, d), x.dtype),
            jax.ShapeDtypeStruct((N, 1), jnp.float32),
        ),
        grid_spec=pltpu.PrefetchScalarGridSpec(
            num_scalar_prefetch=0,
            grid=grid,
            in_specs=[
                pl.BlockSpec(memory_space=pltpu.MemorySpace.SMEM),
                pl.BlockSpec((tile, d), lambda i: (i, 0)),
                pl.BlockSpec((1, d), lambda i: (0, 0)),
                pl.BlockSpec((1, d), lambda i: (0, 0)),
            ],
            out_specs=[
                pl.BlockSpec((tile, d), lambda i: (i, 0)),
                pl.BlockSpec((tile, 1), lambda i: (i, 0)),
            ],
        ),
        compiler_params=pltpu.CompilerParams(
            dimension_semantics=("parallel",),
            vmem_limit_bytes=48 * 1024 * 1024),
    )(scalars, x, wf, u_hat)
    return out, ld


# speedup vs baseline: 1.1246x; 1.1246x over previous
import jax
import jax.numpy as jnp
from jax.experimental import pallas as pl
from jax.experimental.pallas import tpu as pltpu

_LANES = 128
_SUBLANES = 8


def _params(w_row, u_row, b_ref):
    wtu = jnp.sum(w_row * u_row, axis=1, keepdims=True)
    wsq = jnp.sum(w_row * w_row, axis=1, keepdims=True)
    m_wtu = -1.0 + jnp.log1p(jnp.exp(wtu))
    u_hat = jnp.where(wtu < 1.0, u_row + (m_wtu - wtu) * w_row / wsq, u_row)
    wtu_hat = jnp.sum(w_row * u_hat, axis=1, keepdims=True)
    return w_row.T, u_hat.T, b_ref[0], wtu_hat


def _colwise_kernel(b_ref, xt_ref, w_ref, u_ref, out_ref, ld_ref):
    w_col, u_col, b, wtu = _params(w_ref[...].astype(jnp.float32),
                                   u_ref[...].astype(jnp.float32), b_ref)

    xf = xt_ref[...].astype(jnp.float32)
    lin = jnp.sum(xf * w_col, axis=0, keepdims=True) + b
    h = jnp.tanh(lin)
    out_ref[...] = (xf + u_col * h).astype(out_ref.dtype)
    ld_ref[...] = jnp.log(jnp.abs(1.0 + (1.0 - h * h) * wtu))


def _rowwise_kernel(b_ref, x_ref, w_ref, u_ref, out_ref, ld_ref):
    w_col, u_col, b, wtu = _params(w_ref[...].astype(jnp.float32),
                                   u_ref[...].astype(jnp.float32), b_ref)

    xf = x_ref[...].astype(jnp.float32)
    lin = jnp.sum(xf * w_col.T, axis=-1, keepdims=True) + b
    h = jnp.tanh(lin)
    out_ref[...] = (xf + h * u_col.T).astype(out_ref.dtype)
    ld_ref[...] = jnp.log(jnp.abs(1.0 + (1.0 - h * h) * wtu))


def kernel(x, u, w, b):
    N, d = x.shape
    w_row = w.astype(jnp.float32).reshape(1, d)
    u_row = u.astype(jnp.float32).reshape(1, d)
    b_vec = b.astype(jnp.float32).reshape(1)

    if d % _SUBLANES == 0 and N % _LANES == 0:
        xt = x.T

        tl = 65536
        while tl > _LANES and N % tl != 0:
            tl //= 2
        grid = (N // tl,)
        block_bytes = d * tl * 8 + tl * 4
        vmem_limit = int(min(60000 * 1024, 2 * block_bytes + (16 << 20)))

        out_t, ld_row = pl.pallas_call(
            _colwise_kernel,
            out_shape=(
                jax.ShapeDtypeStruct((d, N), x.dtype),
                jax.ShapeDtypeStruct((1, N), jnp.float32),
            ),
            grid_spec=pltpu.PrefetchScalarGridSpec(
                num_scalar_prefetch=0,
                grid=grid,
                in_specs=[
                    pl.BlockSpec(memory_space=pltpu.MemorySpace.SMEM),
                    pl.BlockSpec((d, tl), lambda i: (0, i)),
                    pl.BlockSpec((1, d), lambda i: (0, 0)),
                    pl.BlockSpec((1, d), lambda i: (0, 0)),
                ],
                out_specs=[
                    pl.BlockSpec((d, tl), lambda i: (0, i)),
                    pl.BlockSpec((1, tl), lambda i: (0, i)),
                ],
            ),
            compiler_params=pltpu.CompilerParams(
                dimension_semantics=("parallel",),
                vmem_limit_bytes=vmem_limit),
        )(b_vec, xt, w_row, u_row)

        return out_t.T, ld_row.reshape(N, 1)

    tile = min(N, 4096)
    tile = max(_SUBLANES, (tile // _SUBLANES) * _SUBLANES)
    grid = (pl.cdiv(N, tile),)
    out, ld = pl.pallas_call(
        _rowwise_kernel,
        out_shape=(
            jax.ShapeDtypeStruct((N, d), x.dtype),
            jax.ShapeDtypeStruct((N, 1), jnp.float32),
        ),
        grid_spec=pltpu.PrefetchScalarGridSpec(
            num_scalar_prefetch=0,
            grid=grid,
            in_specs=[
                pl.BlockSpec(memory_space=pltpu.MemorySpace.SMEM),
                pl.BlockSpec((tile, d), lambda i: (i, 0)),
                pl.BlockSpec((1, d), lambda i: (0, 0)),
                pl.BlockSpec((1, d), lambda i: (0, 0)),
            ],
            out_specs=[
                pl.BlockSpec((tile, d), lambda i: (i, 0)),
                pl.BlockSpec((tile, 1), lambda i: (i, 0)),
            ],
        ),
        compiler_params=pltpu.CompilerParams(
            dimension_semantics=("parallel",),
            vmem_limit_bytes=48 * 1024 * 1024),
    )(b_vec, x, w_row, u_row)
    return out, ld
